# baseline (speedup 1.0000x reference)
import functools

import jax
import jax.numpy as jnp
from jax import lax
from jax.experimental import pallas as pl
from jax.experimental.pallas import tpu as pltpu
from jax.experimental.pallas import tpu_sc as plsc

_NUM_CORES = 2
_NUM_SUBCORES = 16
_NUM_WORKERS = _NUM_CORES * _NUM_SUBCORES
_LANES = 16


def _pick_partition(n):
    for w in range(_NUM_WORKERS, 0, -1):
        if n % w == 0 and (n // w) % _LANES == 0:
            return w, n // w
    return None


@functools.lru_cache(maxsize=None)
def _build(n, table_size):
    part = _pick_partition(n)
    if part is None:
        raise ValueError(f"no partition for n={n}")
    num_workers, chunk = part
    mesh = plsc.VectorSubcoreMesh(core_axis_name="c", subcore_axis_name="s")

    @functools.partial(
        pl.kernel,
        mesh=mesh,
        compiler_params=pltpu.CompilerParams(needs_layout_passes=False),
        out_type=jax.ShapeDtypeStruct((n,), jnp.int32),
        scratch_types=[
            pltpu.VMEM((table_size,), jnp.int32),
            pltpu.VMEM((table_size,), jnp.int32),
            pltpu.VMEM((chunk,), jnp.int32),
            pltpu.VMEM((chunk,), jnp.int32),
        ],
    )
    def lookup(nodes_hbm, species_hbm, out_hbm, spec_v, inv_v, in_v, res_v):
        wid = lax.axis_index("s") * _NUM_CORES + lax.axis_index("c")

        @pl.when(wid < num_workers)
        def _():
            base = wid * chunk
            pltpu.sync_copy(species_hbm, spec_v)
            pltpu.sync_copy(nodes_hbm.at[pl.ds(base, chunk)], in_v)
            for j in range(table_size // _LANES):
                sp = spec_v[pl.ds(j * _LANES, _LANES)]
                ids = lax.iota(jnp.int32, _LANES) + j * _LANES
                plsc.store_scatter(inv_v, [sp], ids)

            def body(i, carry):
                off = pl.multiple_of(i * _LANES, _LANES)
                x = in_v[pl.ds(off, _LANES)]
                res_v[pl.ds(off, _LANES)] = plsc.load_gather(inv_v, [x])
                return carry

            lax.fori_loop(0, chunk // _LANES, body, 0)
            pltpu.sync_copy(res_v, out_hbm.at[pl.ds(base, chunk)])

    return lookup


def kernel(node_atomic_numbers, species):
    n = node_atomic_numbers.shape[0]
    return _build(n, species.shape[0])(
        node_atomic_numbers.astype(jnp.int32), species.astype(jnp.int32)
    )
